# baseline (speedup 1.0000x reference)
import functools

import jax
import jax.numpy as jnp
from jax import lax
from jax.experimental import pallas as pl
from jax.experimental.pallas import tpu as pltpu
from jax.experimental.pallas import tpu_sc as plsc

_INFO = plsc.get_sparse_core_info()
_NC, _NS = _INFO.num_cores, _INFO.num_subcores
_NW = _NC * _NS


@functools.partial(jax.jit, static_argnames=("b_per_w", "chunk", "sb"))
def _sc_gather(table, idx, *, b_per_w, chunk, sb):
    n_chunks = b_per_w // chunk
    nsb = n_chunks // sb
    rows_sb = sb * chunk
    B = idx.shape[0]
    D = table.shape[1]
    mesh = plsc.VectorSubcoreMesh(core_axis_name="c", subcore_axis_name="s")

    @functools.partial(
        pl.kernel,
        mesh=mesh,
        out_type=jax.ShapeDtypeStruct((B, D), jnp.float32),
        compiler_params=pltpu.CompilerParams(use_tc_tiling_on_sc=False),
        scratch_types=[
            pltpu.VMEM((b_per_w,), jnp.int32),
            pltpu.VMEM((2, rows_sb, D), jnp.float32),
            [pltpu.SemaphoreType.DMA] * 2,
            [pltpu.SemaphoreType.DMA] * 2,
        ],
    )
    def k(table_hbm, idx_hbm, out_hbm, idx_v, rows_v, gsem, osem):
        wid = lax.axis_index("s") * _NC + lax.axis_index("c")
        base = wid * b_per_w
        pltpu.sync_copy(idx_hbm.at[pl.ds(base, b_per_w)], idx_v)

        def gathers_start(s, p):
            for c in range(sb):
                pltpu.async_copy(
                    table_hbm.at[idx_v.at[pl.ds((s * sb + c) * chunk, chunk)]],
                    rows_v.at[p].at[pl.ds(c * chunk, chunk)],
                    gsem[p],
                )

        def gathers_wait(p):
            pltpu.make_async_copy(
                table_hbm.at[pl.ds(0, rows_sb)], rows_v.at[p], gsem[p]
            ).wait()

        def out_copy(s, p):
            return pltpu.make_async_copy(
                rows_v.at[p], out_hbm.at[pl.ds(base + s * rows_sb, rows_sb)], osem[p]
            )

        gathers_start(0, 0)
        gathers_start(1, 1)

        def group(g, carry):
            for p in range(2):
                s = g * 2 + p
                gathers_wait(p)
                out_copy(s, p).start()
                out_copy(s, p).wait()
                gathers_start(s + 2, p)
            return carry

        lax.fori_loop(0, nsb // 2 - 1, group, 0)

        for p in range(2):
            s = nsb - 2 + p
            gathers_wait(p)
            out_copy(s, p).start()
            out_copy(s, p).wait()

    return k(table, idx)


def kernel(indices, table):
    B = indices.shape[0] * indices.shape[1]
    idx_flat = indices.reshape(B).astype(jnp.int32)
    out = _sc_gather(table, idx_flat, b_per_w=B // _NW, chunk=128, sb=5)
    return out.reshape(indices.shape + (table.shape[1],))
